# baseline (speedup 1.0000x reference)
import functools

import jax
import jax.numpy as jnp
import numpy as np
from jax import lax
from jax.experimental import pallas as pl
from jax.experimental.pallas import tpu as pltpu
from jax.experimental.pallas import tpu_sc as plsc

C = 64
G = 8
B = 4096
P = 49
MOM = 0.1
INV_COUNT = 1.0 / (G * B)

TSPLIT = 25
KSC = P - TSPLIT
PCHUNK = 5
NSTEPS = TSPLIT // PCHUNK

NW = 32
ITEMS = KSC * G
IPW = ITEMS // NW


def _tc_body(x_ref, gm_ref, o_ref):
    blk = x_ref[...]
    part = jnp.sum(blk.reshape(PCHUNK, G, G, B), axis=(2, 3))
    o_ref[0] = (1.0 - MOM) * gm_ref[0] + (MOM * INV_COUNT) * part


def _tc_call(x, gm_t):
    return pl.pallas_call(
        _tc_body,
        grid=(NSTEPS,),
        in_specs=[
            pl.BlockSpec((PCHUNK, C, B), lambda j: (j, 0, 0)),
            pl.BlockSpec((1, PCHUNK, G), lambda j: (j, 0, 0)),
        ],
        out_specs=pl.BlockSpec((1, PCHUNK, G), lambda j: (j, 0, 0)),
        out_shape=jax.ShapeDtypeStruct((NSTEPS, PCHUNK, G), jnp.float32),
    )(x, gm_t)


def _sc_reduce_item(buf):
    def row_fold(k, acc):
        off = pl.multiple_of(k * 16, 16)
        for r in range(8):
            acc = acc + buf[r, pl.ds(off, 16)]
        return acc
    return lax.fori_loop(0, B // 16, row_fold, jnp.zeros((16,), jnp.float32))


def _sc_body(x_hbm, out_hbm, buf0, buf1, res_v, sem0, sem1):
    wid = lax.axis_index("s") * 2 + lax.axis_index("c")
    base = TSPLIT * G + wid * IPW
    bufs = (buf0, buf1)
    sems = (sem0, sem1)

    def copy(j):
        return pltpu.make_async_copy(
            x_hbm.at[base + j], bufs[j % 2], sems[j % 2])

    copy(0).start()
    for j in range(IPW):
        copy(j).wait()
        if j + 1 < IPW:
            copy(j + 1).start()
        res_v[j, :] = _sc_reduce_item(bufs[j % 2])
    pltpu.sync_copy(res_v, out_hbm.at[wid])


_sc_call = functools.partial(
    pl.kernel,
    out_type=jax.ShapeDtypeStruct((NW, IPW, 16), jnp.float32),
    mesh=plsc.VectorSubcoreMesh(core_axis_name="c", subcore_axis_name="s"),
    scratch_types=[
        pltpu.VMEM((G, B), jnp.float32),
        pltpu.VMEM((G, B), jnp.float32),
        pltpu.VMEM((IPW, 16), jnp.float32),
        pltpu.SemaphoreType.DMA,
        pltpu.SemaphoreType.DMA,
    ],
)(_sc_body)


def kernel(query_attention_maps, group_memory):
    x = jnp.transpose(query_attention_maps, (0, 3, 4, 1, 2)).reshape(P, C, B)
    x_items = x.reshape(P * G, G, B)
    gm = group_memory.reshape(G, P)
    gm_t = gm.T[:TSPLIT].reshape(NSTEPS, PCHUNK, G)

    tc_res = _tc_call(x, gm_t)
    sc_raw = _sc_call(x_items)

    sc_sums = sc_raw.sum(axis=2).reshape(KSC, G).T
    sc_res = (1.0 - MOM) * gm[:, TSPLIT:] + (MOM * INV_COUNT) * sc_sums
    out = jnp.concatenate([tc_res.reshape(TSPLIT, G).T, sc_res], axis=1)
    return out.reshape(G, 1, 7, 7)


# speedup vs baseline: 2.0060x; 2.0060x over previous
import jax
import jax.numpy as jnp
from jax.experimental import pallas as pl

C = 64
G = 8
B = 4096
P = 49
PCHUNK = 7
NSTEPS = P // PCHUNK
NSPLIT = 4
CS = C // NSPLIT
GS = G // NSPLIT
MOM = 0.1
INV_COUNT = 1.0 / (G * B)


def _body(x0_ref, x1_ref, x2_ref, x3_ref, gm_ref, o_ref):
    parts = []
    for r in (x0_ref, x1_ref, x2_ref, x3_ref):
        blk = r[...]
        parts.append(jnp.sum(blk.reshape(PCHUNK, GS, G, B), axis=(2, 3)))
    part = jnp.concatenate(parts, axis=1)
    o_ref[0] = (1.0 - MOM) * gm_ref[0] + (MOM * INV_COUNT) * part


def kernel(query_attention_maps, group_memory):
    x = jnp.transpose(query_attention_maps, (0, 3, 4, 1, 2)).reshape(P, C, B)
    gm_t = group_memory.reshape(G, P).T.reshape(NSTEPS, PCHUNK, G)
    xspec = lambda s: pl.BlockSpec((PCHUNK, CS, B), lambda j, s=s: (j, s, 0))
    res = pl.pallas_call(
        _body,
        grid=(NSTEPS,),
        in_specs=[xspec(0), xspec(1), xspec(2), xspec(3),
                  pl.BlockSpec((1, PCHUNK, G), lambda j: (j, 0, 0))],
        out_specs=pl.BlockSpec((1, PCHUNK, G), lambda j: (j, 0, 0)),
        out_shape=jax.ShapeDtypeStruct((NSTEPS, PCHUNK, G), jnp.float32),
    )(x, x, x, x, gm_t)
    return res.reshape(P, G).T.reshape(G, 1, 7, 7)
